# baseline (speedup 1.0000x reference)
import functools
import math

import jax
import jax.numpy as jnp
from jax.experimental import pallas as pl
from jax.experimental.pallas import tpu as pltpu

_K = 32
_QB = 512
_SEARCH_BITS = 20


def _topk_attn_kernel(q_ref, k_ref, v_ref, o_ref, *, qi):
    QB = q_ref.shape[1]
    L = k_ref.shape[1]
    D = q_ref.shape[2]
    scale = 1.0 / math.sqrt(D)

    G = q_ref.shape[0]

    q = q_ref[...]
    k = k_ref[...]
    s = jax.lax.dot_general(
        q, k, (((2,), (2,)), ((0,), (0,))), preferred_element_type=jnp.float32
    ) * scale

    row = qi * QB + jax.lax.broadcasted_iota(jnp.int32, (G, QB, L), 1)
    col = jax.lax.broadcasted_iota(jnp.int32, (G, QB, L), 2)
    s = jnp.where(col > row, -jnp.inf, s)

    m = jnp.max(s, axis=2, keepdims=True)
    w = s - m

    thr = jnp.full((G, QB, 1), -8.0, jnp.float32)
    step = 4.0
    for _ in range(_SEARCH_BITS):
        cand = thr + step
        cnt = jnp.sum((w >= cand).astype(jnp.float32), axis=2, keepdims=True)
        thr = jnp.where(cnt >= float(_K), cand, thr)
        step *= 0.5

    p = jnp.where(w >= thr, jnp.exp(w), 0.0)
    v_aug = jnp.concatenate(
        [v_ref[...], jnp.ones((G, L, 128), jnp.float32)], axis=2
    )
    o2 = jax.lax.dot_general(
        p, v_aug, (((2,), (1,)), ((0,), (0,))), preferred_element_type=jnp.float32
    )
    o_ref[...] = o2[:, :, :D] / o2[:, :, D : D + 1]


def kernel(q, k, v, k_sparse):
    B, H, T, D = q.shape
    QB = _QB
    BH = B * H
    NQ = T // QB
    q3 = q.reshape(BH, T, D)
    k3 = k.reshape(BH, T, D)
    v3 = v.reshape(BH, T, D)

    G = 4
    outs = []
    for qi in range(NQ):
        L = (qi + 1) * QB
        out_qi = pl.pallas_call(
            functools.partial(_topk_attn_kernel, qi=qi),
            grid=(BH // G,),
            in_specs=[
                pl.BlockSpec((G, QB, D), lambda bh, qi=qi: (bh, qi, 0)),
                pl.BlockSpec((G, L, D), lambda bh: (bh, 0, 0)),
                pl.BlockSpec((G, L, D), lambda bh: (bh, 0, 0)),
            ],
            out_specs=pl.BlockSpec((G, QB, D), lambda bh: (bh, 0, 0)),
            out_shape=jax.ShapeDtypeStruct((BH, QB, D), jnp.float32),
            compiler_params=pltpu.CompilerParams(
                dimension_semantics=("parallel",),
            ),
        )(q3, k3, v3)
        outs.append(out_qi)
    out = jnp.concatenate(outs, axis=1)
    return out.reshape(B, H, T, D)


# speedup vs baseline: 1.1401x; 1.1401x over previous
import functools
import math

import jax
import jax.numpy as jnp
from jax.experimental import pallas as pl
from jax.experimental.pallas import tpu as pltpu

_K = 32
_QB = 512
_SEARCH_BITS = 20


def _topk_attn_kernel(q_ref, k_ref, v_ref, o_ref, *, qi):
    QB = q_ref.shape[1]
    L = k_ref.shape[1]
    D = q_ref.shape[2]
    scale = 1.0 / math.sqrt(D)

    G = q_ref.shape[0]

    q = q_ref[...]
    k = k_ref[...]
    s = jax.lax.dot_general(
        q, k, (((2,), (2,)), ((0,), (0,))), preferred_element_type=jnp.float32
    ) * scale

    row = qi * QB + jax.lax.broadcasted_iota(jnp.int32, (G, QB, L), 1)
    col = jax.lax.broadcasted_iota(jnp.int32, (G, QB, L), 2)
    s = jnp.where(col > row, -jnp.inf, s)

    m = jnp.max(s, axis=2, keepdims=True)
    w = s - m

    thr = jnp.full((G, QB, 1), -8.0, jnp.float32)
    step = 4.0
    for _ in range(_SEARCH_BITS):
        cand = thr + step
        cnt = jnp.sum((w >= cand).astype(jnp.float32), axis=2, keepdims=True)
        thr = jnp.where(cnt >= float(_K), cand, thr)
        step *= 0.5

    p = jnp.where(w >= thr, jnp.exp(w), 0.0)
    v_aug = jnp.concatenate(
        [v_ref[...], jnp.ones((G, L, 128), jnp.float32)], axis=2
    )
    o2 = jax.lax.dot_general(
        p, v_aug, (((2,), (1,)), ((0,), (0,))), preferred_element_type=jnp.float32
    )
    o_ref[...] = o2[:, :, :D] / o2[:, :, D : D + 1]


def kernel(q, k, v, k_sparse):
    B, H, T, D = q.shape
    QB = _QB
    BH = B * H
    NQ = T // QB
    q3 = q.reshape(BH, T, D)
    k3 = k.reshape(BH, T, D)
    v3 = v.reshape(BH, T, D)

    G = 2
    outs = []
    for qi in range(NQ):
        L = (qi + 1) * QB
        out_qi = pl.pallas_call(
            functools.partial(_topk_attn_kernel, qi=qi),
            grid=(BH // G,),
            in_specs=[
                pl.BlockSpec((G, QB, D), lambda bh, qi=qi: (bh, qi, 0)),
                pl.BlockSpec((G, L, D), lambda bh: (bh, 0, 0)),
                pl.BlockSpec((G, L, D), lambda bh: (bh, 0, 0)),
            ],
            out_specs=pl.BlockSpec((G, QB, D), lambda bh: (bh, 0, 0)),
            out_shape=jax.ShapeDtypeStruct((BH, QB, D), jnp.float32),
            compiler_params=pltpu.CompilerParams(
                dimension_semantics=("parallel",),
            ),
        )(q3, k3, v3)
        outs.append(out_qi)
    out = jnp.concatenate(outs, axis=1)
    return out.reshape(B, H, T, D)


# speedup vs baseline: 1.1885x; 1.0424x over previous
import functools
import math

import jax
import jax.numpy as jnp
from jax.experimental import pallas as pl
from jax.experimental.pallas import tpu as pltpu

_K = 32
_QB = 512
_SEARCH_BITS = 19


def _topk_attn_kernel(q_ref, k_ref, v_ref, o_ref, *, qi):
    QB = q_ref.shape[1]
    L = k_ref.shape[1]
    D = q_ref.shape[2]
    scale = 1.0 / math.sqrt(D)

    G = q_ref.shape[0]

    q = q_ref[...]
    k = k_ref[...]
    s = jax.lax.dot_general(
        q, k, (((2,), (2,)), ((0,), (0,))), preferred_element_type=jnp.float32
    ) * scale

    row = qi * QB + jax.lax.broadcasted_iota(jnp.int32, (G, QB, 1), 1)
    col = jax.lax.broadcasted_iota(jnp.int32, (G, QB, L), 2)
    s = jnp.where(col > row, -jnp.inf, s)

    m = jnp.max(s, axis=2, keepdims=True)
    w = s - m

    thr = jnp.full((G, QB, 1), -8.0, jnp.float32)
    step = 4.0
    for _ in range(_SEARCH_BITS):
        cand = thr + step
        cnt = jnp.sum((w >= cand).astype(jnp.float32), axis=2, keepdims=True)
        thr = jnp.where(cnt >= float(_K), cand, thr)
        step *= 0.5

    p = jnp.where(w >= thr, jnp.exp(w), 0.0)
    v_aug = jnp.concatenate(
        [v_ref[...], jnp.ones((G, L, 128), jnp.float32)], axis=2
    )
    o2 = jax.lax.dot_general(
        p, v_aug, (((2,), (1,)), ((0,), (0,))), preferred_element_type=jnp.float32
    )
    o_ref[...] = o2[:, :, :D] / o2[:, :, D : D + 1]


def kernel(q, k, v, k_sparse):
    B, H, T, D = q.shape
    QB = _QB
    BH = B * H
    NQ = T // QB
    q3 = q.reshape(BH, T, D)
    k3 = k.reshape(BH, T, D)
    v3 = v.reshape(BH, T, D)

    G = 2
    outs = []
    for qi in range(NQ):
        L = (qi + 1) * QB
        out_qi = pl.pallas_call(
            functools.partial(_topk_attn_kernel, qi=qi),
            grid=(BH // G,),
            in_specs=[
                pl.BlockSpec((G, QB, D), lambda bh, qi=qi: (bh, qi, 0)),
                pl.BlockSpec((G, L, D), lambda bh: (bh, 0, 0)),
                pl.BlockSpec((G, L, D), lambda bh: (bh, 0, 0)),
            ],
            out_specs=pl.BlockSpec((G, QB, D), lambda bh: (bh, 0, 0)),
            out_shape=jax.ShapeDtypeStruct((BH, QB, D), jnp.float32),
            compiler_params=pltpu.CompilerParams(
                dimension_semantics=("parallel",),
            ),
        )(q3, k3, v3)
        outs.append(out_qi)
    out = jnp.concatenate(outs, axis=1)
    return out.reshape(B, H, T, D)
